# baseline (speedup 1.0000x reference)
import jax
import jax.numpy as jnp
from jax.experimental import pallas as pl
from jax.experimental.pallas import tpu as pltpu

_BIG = 1e30
_LANE = 128
_P = 8


def _matcher_body(pred_ref, tgt_ref, out_ref, cost_ref,
                  u_ref, v_ref, r4c_ref, c4r_ref, c4rcol_ref,
                  sr_ref, rowminv_ref, sc_ref, spc_ref, path_ref, minv_ref):
    P = cost_ref.shape[0]
    G = cost_ref.shape[2]
    M = cost_ref.shape[1]

    for p in range(P):
        tx1 = tgt_ref[p, :, 0:1]
        ty1 = tgt_ref[p, :, 1:2]
        tx2 = tgt_ref[p, :, 2:3]
        ty2 = tgt_ref[p, :, 3:4]
        area_t = (tx2 - tx1) * (ty2 - ty1)
        for g in range(G):
            px1 = pred_ref[p, 0, g, :]
            py1 = pred_ref[p, 1, g, :]
            px2 = pred_ref[p, 2, g, :]
            py2 = pred_ref[p, 3, g, :]
            c_l1 = ((jnp.abs(px1 - tx1) + jnp.abs(py1 - ty1))
                    + jnp.abs(px2 - tx2)) + jnp.abs(py2 - ty2)
            area_p = (px2 - px1) * (py2 - py1)
            ltx = jnp.maximum(px1, tx1)
            lty = jnp.maximum(py1, ty1)
            rbx = jnp.minimum(px2, tx2)
            rby = jnp.minimum(py2, ty2)
            iw = jnp.maximum(rbx - ltx, 0.0)
            ih = jnp.maximum(rby - lty, 0.0)
            inter = iw * ih
            union = (area_p + area_t) - inter
            iou = inter / (union + 1e-8)
            elx = jnp.minimum(px1, tx1)
            ely = jnp.minimum(py1, ty1)
            erx = jnp.maximum(px2, tx2)
            ery = jnp.maximum(py2, ty2)
            ew = jnp.maximum(erx - elx, 0.0)
            eh = jnp.maximum(ery - ely, 0.0)
            earea = ew * eh
            giou = iou - (earea - union) / (earea + 1e-8)
            cost_ref[p, :, g, :] = c_l1 - giou

    riota = jax.lax.broadcasted_iota(jnp.int32, (1, M), 1)
    citer = jax.lax.broadcasted_iota(jnp.int32, (M, 1), 0)
    fiota = (jax.lax.broadcasted_iota(jnp.int32, (G, _LANE), 0) * _LANE
             + jax.lax.broadcasted_iota(jnp.int32, (G, _LANE), 1))

    u_ref[...] = jnp.zeros((P, M), jnp.float32)
    v_ref[...] = jnp.zeros((P, G, _LANE), jnp.float32)
    r4c_ref[...] = jnp.full((P, G, _LANE), -1, jnp.int32)
    c4r_ref[...] = jnp.full((P, M), -1, jnp.int32)
    c4rcol_ref[...] = jnp.full((P, M, 1), -1, jnp.int32)

    def outer(i, _):
        sr_ref[...] = jnp.zeros((P, M), jnp.int32)
        rowminv_ref[...] = jnp.zeros((P, M), jnp.float32)
        sc_ref[...] = jnp.zeros((P, G, _LANE), jnp.int32)
        spc_ref[...] = jnp.full((P, G, _LANE), _BIG, jnp.float32)
        path_ref[...] = jnp.zeros((P, G, _LANE), jnp.int32)
        minv_ref[...] = jnp.zeros((P, 1), jnp.float32)

        def sp_cond(s):
            m = s[P]
            for p in range(1, P):
                m = jnp.minimum(m, s[P + p])
            return m < 0

        def sp_body(s):
            new_cur, new_sink = [], []
            for p in range(P):
                cur, sink = s[p], s[P + p]
                active = sink < 0
                minv = minv_ref[p:p + 1, :]
                u = u_ref[p:p + 1, :]
                curmask = (riota == cur) & active
                sr_ref[p:p + 1, :] = jnp.where(curmask, 1, sr_ref[p:p + 1, :])
                ucur = jnp.sum(jnp.where(curmask, u, 0.0), axis=1, keepdims=True)
                c4rcur = jnp.sum(jnp.where(curmask, c4r_ref[p:p + 1, :], 0),
                                 axis=1, keepdims=True)
                pathpk = (cur + 1) * 4096 + (c4rcur + 1)
                crow = cost_ref[p, cur]
                red = ((minv + crow) - ucur) - v_ref[p]
                sc = sc_ref[p]
                spc = spc_ref[p]
                better = active & (sc == 0) & (red < spc)
                spc = jnp.where(better, red, spc)
                spc_ref[p] = spc
                path_ref[p] = jnp.where(better, pathpk, path_ref[p])
                masked = jnp.where(sc != 0, _BIG, spc)
                minv_new = jnp.min(jnp.min(masked, axis=1, keepdims=True),
                                   axis=0, keepdims=True)
                minv_ref[p:p + 1, :] = jnp.where(active, minv_new, minv)
                key = jnp.where(masked == minv_new,
                                fiota * 256 + (r4c_ref[p] + 1),
                                jnp.int32(1 << 30))
                kmin = jnp.min(key)
                j = kmin >> 8
                r4cj = (kmin & 255) - 1
                sc_ref[p] = jnp.where(active & (fiota == j), 1, sc)
                unassigned = r4cj < 0
                new_sink.append(jnp.where(active,
                                          jnp.where(unassigned, j, jnp.int32(-1)),
                                          sink))
                new_cur.append(jnp.where(active & ~unassigned, r4cj, cur))
                rowminv_ref[p:p + 1, :] = jnp.where(
                    (riota == r4cj) & active, minv_new, rowminv_ref[p:p + 1, :])
            return (*new_cur, *new_sink)

        init = tuple([i] * P) + tuple([jnp.int32(-1)] * P)
        fin = jax.lax.while_loop(sp_cond, sp_body, init)
        sinks = fin[P:]

        minv_all = minv_ref[...]
        u = u_ref[...]
        u = jnp.where(riota == i, u + minv_all, u)
        rmask = (sr_ref[...] != 0) & (riota != i)
        u_ref[...] = jnp.where(rmask, (u + minv_all) - rowminv_ref[...], u)
        for p in range(P):
            minv = minv_ref[p:p + 1, :]
            sc = sc_ref[p] != 0
            v = v_ref[p]
            v_ref[p] = jnp.where(sc, (v + spc_ref[p]) - minv, v)

        def aug_cond(s):
            m = s[P]
            for p in range(1, P):
                m = jnp.minimum(m, s[P + p])
            return m == 0

        def aug_body(s):
            new_j, new_done = [], []
            for p in range(P):
                j, done = s[p], s[P + p]
                active = done == 0
                pk = jnp.sum(jnp.where((fiota == j) & active, path_ref[p], 0))
                ii = (pk >> 12) - 1
                nj = (pk & 4095) - 1
                r4c_ref[p] = jnp.where(active & (fiota == j), ii, r4c_ref[p])
                c4r_ref[p:p + 1, :] = jnp.where(
                    (riota == ii) & active, j, c4r_ref[p:p + 1, :])
                c4rcol_ref[p] = jnp.where(
                    (citer == ii) & active, j, c4rcol_ref[p])
                new_j.append(jnp.where(active, nj, j))
                new_done.append(jnp.where(active & (ii != i), jnp.int32(0),
                                          jnp.int32(1)))
            return (*new_j, *new_done)

        jax.lax.while_loop(aug_cond, aug_body,
                           tuple(sinks) + tuple([jnp.int32(0)] * P))
        return 0

    jax.lax.fori_loop(0, M, outer, 0)

    for p in range(P):
        c4r = c4r_ref[p:p + 1, :]
        c4rcol = c4rcol_ref[p]
        less = c4r < c4rcol
        rank = jnp.sum(less.astype(jnp.int32), axis=1, keepdims=True)
        onehot = rank == riota
        pred_sorted = jnp.sum(jnp.where(onehot, c4rcol, 0), axis=0, keepdims=True)
        tgt_sorted = jnp.sum(jnp.where(onehot, citer, 0), axis=0, keepdims=True)
        out_ref[p, 0:1, :] = pred_sorted
        out_ref[p, 1:2, :] = tgt_sorted


def kernel(out_boxes, tgt_boxes):
    B, N, _ = out_boxes.shape
    M = tgt_boxes.shape[1]
    G = N // _LANE
    pred = jnp.transpose(out_boxes, (0, 2, 1)).reshape(B, 4, G, _LANE)
    pred = jnp.pad(pred, ((0, 0), (0, 4), (0, 0), (0, 0)))
    tgt = jnp.pad(tgt_boxes, ((0, 0), (0, 0), (0, 4)))
    out = pl.pallas_call(
        _matcher_body,
        grid=(B // _P,),
        in_specs=[pl.BlockSpec((_P, 8, G, _LANE), lambda b: (b, 0, 0, 0)),
                  pl.BlockSpec((_P, M, 8), lambda b: (b, 0, 0))],
        out_specs=pl.BlockSpec((_P, 8, M), lambda b: (b, 0, 0)),
        out_shape=jax.ShapeDtypeStruct((B, 8, M), jnp.int32),
        scratch_shapes=[pltpu.VMEM((_P, M, G, _LANE), jnp.float32),
                        pltpu.VMEM((_P, M), jnp.float32),
                        pltpu.VMEM((_P, G, _LANE), jnp.float32),
                        pltpu.VMEM((_P, G, _LANE), jnp.int32),
                        pltpu.VMEM((_P, M), jnp.int32),
                        pltpu.VMEM((_P, M, 1), jnp.int32),
                        pltpu.VMEM((_P, M), jnp.int32),
                        pltpu.VMEM((_P, M), jnp.float32),
                        pltpu.VMEM((_P, G, _LANE), jnp.int32),
                        pltpu.VMEM((_P, G, _LANE), jnp.float32),
                        pltpu.VMEM((_P, G, _LANE), jnp.int32),
                        pltpu.VMEM((_P, 1), jnp.float32)],
        compiler_params=pltpu.CompilerParams(
            dimension_semantics=("parallel",)),
    )(pred, tgt)
    return out[:, :2, :]


# speedup vs baseline: 2.2358x; 2.2358x over previous
import jax
import jax.numpy as jnp
from jax.experimental import pallas as pl
from jax.experimental.pallas import tpu as pltpu

_BIG = 1e30
_LANE = 128
_P = 4


def _matcher_body(pred_ref, tgt_ref, out_ref, cost_ref,
                  u_ref, v_ref, r4c_ref, c4r_ref, c4rcol_ref,
                  sr_ref, rowminv_ref, sc_ref, spc_ref, path_ref, minv_ref):
    P = cost_ref.shape[0]
    G = cost_ref.shape[2]
    M = cost_ref.shape[1]

    for p in range(P):
        tx1 = tgt_ref[p, :, 0:1]
        ty1 = tgt_ref[p, :, 1:2]
        tx2 = tgt_ref[p, :, 2:3]
        ty2 = tgt_ref[p, :, 3:4]
        area_t = (tx2 - tx1) * (ty2 - ty1)
        for g in range(G):
            px1 = pred_ref[p, 0, g, :]
            py1 = pred_ref[p, 1, g, :]
            px2 = pred_ref[p, 2, g, :]
            py2 = pred_ref[p, 3, g, :]
            c_l1 = ((jnp.abs(px1 - tx1) + jnp.abs(py1 - ty1))
                    + jnp.abs(px2 - tx2)) + jnp.abs(py2 - ty2)
            area_p = (px2 - px1) * (py2 - py1)
            ltx = jnp.maximum(px1, tx1)
            lty = jnp.maximum(py1, ty1)
            rbx = jnp.minimum(px2, tx2)
            rby = jnp.minimum(py2, ty2)
            iw = jnp.maximum(rbx - ltx, 0.0)
            ih = jnp.maximum(rby - lty, 0.0)
            inter = iw * ih
            union = (area_p + area_t) - inter
            iou = inter / (union + 1e-8)
            elx = jnp.minimum(px1, tx1)
            ely = jnp.minimum(py1, ty1)
            erx = jnp.maximum(px2, tx2)
            ery = jnp.maximum(py2, ty2)
            ew = jnp.maximum(erx - elx, 0.0)
            eh = jnp.maximum(ery - ely, 0.0)
            earea = ew * eh
            giou = iou - (earea - union) / (earea + 1e-8)
            cost_ref[p, :, g, :] = c_l1 - giou

    riota = jax.lax.broadcasted_iota(jnp.int32, (1, M), 1)
    citer = jax.lax.broadcasted_iota(jnp.int32, (M, 1), 0)
    fiota = (jax.lax.broadcasted_iota(jnp.int32, (G, _LANE), 0) * _LANE
             + jax.lax.broadcasted_iota(jnp.int32, (G, _LANE), 1))

    u_ref[...] = jnp.zeros((P, M), jnp.float32)
    v_ref[...] = jnp.zeros((P, G, _LANE), jnp.float32)
    r4c_ref[...] = jnp.full((P, G, _LANE), -1, jnp.int32)
    c4r_ref[...] = jnp.full((P, M), -1, jnp.int32)
    c4rcol_ref[...] = jnp.full((P, M, 1), -1, jnp.int32)

    def outer(i, _):
        sr_ref[...] = jnp.zeros((P, M), jnp.int32)
        rowminv_ref[...] = jnp.zeros((P, M), jnp.float32)
        sc_ref[...] = jnp.zeros((P, G, _LANE), jnp.int32)
        spc_ref[...] = jnp.full((P, G, _LANE), _BIG, jnp.float32)
        path_ref[...] = jnp.zeros((P, G, _LANE), jnp.int32)
        minv_ref[...] = jnp.zeros((P, 1), jnp.float32)

        def sp_cond(s):
            m = s[P]
            for p in range(1, P):
                m = jnp.minimum(m, s[P + p])
            return m < 0

        def sp_body(s):
            A = []
            for p in range(P):
                cur, sink = s[p], s[P + p]
                active = sink < 0
                curmask = (riota == cur) & active
                ucur = jnp.sum(jnp.where(curmask, u_ref[p:p + 1, :], 0.0),
                               axis=1, keepdims=True)
                c4rcur = jnp.sum(jnp.where(curmask, c4r_ref[p:p + 1, :], 0),
                                 axis=1, keepdims=True)
                crow = cost_ref[p, cur]
                A.append((cur, sink, active, curmask, ucur, c4rcur, crow))
            B = []
            for p in range(P):
                cur, sink, active, curmask, ucur, c4rcur, crow = A[p]
                minv = minv_ref[p:p + 1, :]
                pathpk = (cur + 1) * 4096 + (c4rcur + 1)
                red = ((minv + crow) - ucur) - v_ref[p]
                sc = sc_ref[p]
                spc = spc_ref[p]
                better = active & (sc == 0) & (red < spc)
                spc_new = jnp.where(better, red, spc)
                path_new = jnp.where(better, pathpk, path_ref[p])
                masked = jnp.where(sc != 0, _BIG, spc_new)
                minv_row = jnp.min(masked, axis=0, keepdims=True)
                minv_new = jnp.min(minv_row, axis=1, keepdims=True)
                B.append((sc, masked, minv_new, spc_new, path_new, minv))
            K = []
            for p in range(P):
                sc, masked, minv_new, spc_new, path_new, minv = B[p]
                key = jnp.where(masked == minv_new,
                                fiota * 256 + (r4c_ref[p] + 1),
                                jnp.int32(1 << 30))
                kmin = jnp.min(jnp.min(key, axis=0, keepdims=True))
                K.append(kmin)
            new_cur, new_sink, stores = [], [], []
            for p in range(P):
                cur, sink, active, curmask, ucur, c4rcur, crow = A[p]
                sc, masked, minv_new, spc_new, path_new, minv = B[p]
                kmin = K[p]
                j = kmin >> 8
                r4cj = (kmin & 255) - 1
                sr_new = jnp.where(curmask, 1, sr_ref[p:p + 1, :])
                sc_new = jnp.where(active & (fiota == j), 1, sc)
                unassigned = r4cj < 0
                new_sink.append(jnp.where(active,
                                          jnp.where(unassigned, j, jnp.int32(-1)),
                                          sink))
                new_cur.append(jnp.where(active & ~unassigned, r4cj, cur))
                rowminv_new = jnp.where(
                    (riota == r4cj) & active, minv_new, rowminv_ref[p:p + 1, :])
                minv_keep = jnp.where(active, minv_new, minv)
                stores.append((sr_new, spc_new, path_new, sc_new,
                               rowminv_new, minv_keep))
            for p in range(P):
                sr_new, spc_new, path_new, sc_new, rowminv_new, minv_keep = stores[p]
                sr_ref[p:p + 1, :] = sr_new
                spc_ref[p] = spc_new
                path_ref[p] = path_new
                sc_ref[p] = sc_new
                rowminv_ref[p:p + 1, :] = rowminv_new
                minv_ref[p:p + 1, :] = minv_keep
            return (*new_cur, *new_sink)

        init = tuple([i] * P) + tuple([jnp.int32(-1)] * P)
        fin = jax.lax.while_loop(sp_cond, sp_body, init)
        sinks = fin[P:]

        minv_all = minv_ref[...]
        u = u_ref[...]
        u = jnp.where(riota == i, u + minv_all, u)
        rmask = (sr_ref[...] != 0) & (riota != i)
        u_ref[...] = jnp.where(rmask, (u + minv_all) - rowminv_ref[...], u)
        v_new = [jnp.where(sc_ref[p] != 0,
                           (v_ref[p] + spc_ref[p]) - minv_ref[p:p + 1, :],
                           v_ref[p]) for p in range(P)]
        for p in range(P):
            v_ref[p] = v_new[p]

        def aug_cond(s):
            m = s[P]
            for p in range(1, P):
                m = jnp.minimum(m, s[P + p])
            return m == 0

        def aug_body(s):
            pks = []
            for p in range(P):
                j, done = s[p], s[P + p]
                active = done == 0
                pkrow = jnp.sum(jnp.where((fiota == j) & active,
                                          path_ref[p], 0),
                                axis=0, keepdims=True)
                pks.append((j, done, active, jnp.sum(pkrow)))
            new_j, new_done, stores = [], [], []
            for p in range(P):
                j, done, active, pk = pks[p]
                ii = (pk >> 12) - 1
                nj = (pk & 4095) - 1
                r4c_new = jnp.where(active & (fiota == j), ii, r4c_ref[p])
                c4r_new = jnp.where((riota == ii) & active, j,
                                    c4r_ref[p:p + 1, :])
                c4rcol_new = jnp.where((citer == ii) & active, j,
                                       c4rcol_ref[p])
                new_j.append(jnp.where(active, nj, j))
                new_done.append(jnp.where(active & (ii != i), jnp.int32(0),
                                          jnp.int32(1)))
                stores.append((r4c_new, c4r_new, c4rcol_new))
            for p in range(P):
                r4c_new, c4r_new, c4rcol_new = stores[p]
                r4c_ref[p] = r4c_new
                c4r_ref[p:p + 1, :] = c4r_new
                c4rcol_ref[p] = c4rcol_new
            return (*new_j, *new_done)

        jax.lax.while_loop(aug_cond, aug_body,
                           tuple(sinks) + tuple([jnp.int32(0)] * P))
        return 0

    jax.lax.fori_loop(0, M, outer, 0)

    for p in range(P):
        c4r = c4r_ref[p:p + 1, :]
        c4rcol = c4rcol_ref[p]
        less = c4r < c4rcol
        rank = jnp.sum(less.astype(jnp.int32), axis=1, keepdims=True)
        onehot = rank == riota
        pred_sorted = jnp.sum(jnp.where(onehot, c4rcol, 0), axis=0, keepdims=True)
        tgt_sorted = jnp.sum(jnp.where(onehot, citer, 0), axis=0, keepdims=True)
        out_ref[p, 0:1, :] = pred_sorted
        out_ref[p, 1:2, :] = tgt_sorted


def kernel(out_boxes, tgt_boxes):
    B, N, _ = out_boxes.shape
    M = tgt_boxes.shape[1]
    G = N // _LANE
    pred = jnp.transpose(out_boxes, (0, 2, 1)).reshape(B, 4, G, _LANE)
    pred = jnp.pad(pred, ((0, 0), (0, 4), (0, 0), (0, 0)))
    tgt = jnp.pad(tgt_boxes, ((0, 0), (0, 0), (0, 4)))
    out = pl.pallas_call(
        _matcher_body,
        grid=(B // _P,),
        in_specs=[pl.BlockSpec((_P, 8, G, _LANE), lambda b: (b, 0, 0, 0)),
                  pl.BlockSpec((_P, M, 8), lambda b: (b, 0, 0))],
        out_specs=pl.BlockSpec((_P, 8, M), lambda b: (b, 0, 0)),
        out_shape=jax.ShapeDtypeStruct((B, 8, M), jnp.int32),
        scratch_shapes=[pltpu.VMEM((_P, M, G, _LANE), jnp.float32),
                        pltpu.VMEM((_P, M), jnp.float32),
                        pltpu.VMEM((_P, G, _LANE), jnp.float32),
                        pltpu.VMEM((_P, G, _LANE), jnp.int32),
                        pltpu.VMEM((_P, M), jnp.int32),
                        pltpu.VMEM((_P, M, 1), jnp.int32),
                        pltpu.VMEM((_P, M), jnp.int32),
                        pltpu.VMEM((_P, M), jnp.float32),
                        pltpu.VMEM((_P, G, _LANE), jnp.int32),
                        pltpu.VMEM((_P, G, _LANE), jnp.float32),
                        pltpu.VMEM((_P, G, _LANE), jnp.int32),
                        pltpu.VMEM((_P, 1), jnp.float32)],
        compiler_params=pltpu.CompilerParams(
            dimension_semantics=("parallel",)),
    )(pred, tgt)
    return out[:, :2, :]


# speedup vs baseline: 3.3136x; 1.4821x over previous
import jax
import jax.numpy as jnp
from jax.experimental import pallas as pl
from jax.experimental.pallas import tpu as pltpu

_BIG = 1e30
_LANE = 128
_P = 8


def _matcher_body(pred_ref, tgt_ref, out_ref, cost_ref,
                  u_ref, v_ref, r4c_ref, c4r_ref, c4rcol_ref,
                  sr_ref, rowminv_ref, sc_ref, spc_ref, path_ref, minv_ref):
    P = cost_ref.shape[0]
    G = cost_ref.shape[2]
    M = cost_ref.shape[1]

    for p in range(P):
        tx1 = tgt_ref[p, :, 0:1]
        ty1 = tgt_ref[p, :, 1:2]
        tx2 = tgt_ref[p, :, 2:3]
        ty2 = tgt_ref[p, :, 3:4]
        area_t = (tx2 - tx1) * (ty2 - ty1)
        for g in range(G):
            px1 = pred_ref[p, 0, g, :]
            py1 = pred_ref[p, 1, g, :]
            px2 = pred_ref[p, 2, g, :]
            py2 = pred_ref[p, 3, g, :]
            c_l1 = ((jnp.abs(px1 - tx1) + jnp.abs(py1 - ty1))
                    + jnp.abs(px2 - tx2)) + jnp.abs(py2 - ty2)
            area_p = (px2 - px1) * (py2 - py1)
            ltx = jnp.maximum(px1, tx1)
            lty = jnp.maximum(py1, ty1)
            rbx = jnp.minimum(px2, tx2)
            rby = jnp.minimum(py2, ty2)
            iw = jnp.maximum(rbx - ltx, 0.0)
            ih = jnp.maximum(rby - lty, 0.0)
            inter = iw * ih
            union = (area_p + area_t) - inter
            iou = inter / (union + 1e-8)
            elx = jnp.minimum(px1, tx1)
            ely = jnp.minimum(py1, ty1)
            erx = jnp.maximum(px2, tx2)
            ery = jnp.maximum(py2, ty2)
            ew = jnp.maximum(erx - elx, 0.0)
            eh = jnp.maximum(ery - ely, 0.0)
            earea = ew * eh
            giou = iou - (earea - union) / (earea + 1e-8)
            cost_ref[p, :, g, :] = c_l1 - giou

    riota = jax.lax.broadcasted_iota(jnp.int32, (1, M), 1)
    citer = jax.lax.broadcasted_iota(jnp.int32, (M, 1), 0)
    fiota = (jax.lax.broadcasted_iota(jnp.int32, (G, _LANE), 0) * _LANE
             + jax.lax.broadcasted_iota(jnp.int32, (G, _LANE), 1))

    u_ref[...] = jnp.zeros((P, M), jnp.float32)
    v_ref[...] = jnp.zeros((P, G, _LANE), jnp.float32)
    r4c_ref[...] = jnp.full((P, G, _LANE), -1, jnp.int32)
    c4r_ref[...] = jnp.full((P, M), -1, jnp.int32)
    c4rcol_ref[...] = jnp.full((P, M, 1), -1, jnp.int32)

    def outer(i, _):
        sr_ref[...] = jnp.zeros((P, M), jnp.int32)
        rowminv_ref[...] = jnp.zeros((P, M), jnp.float32)
        sc_ref[...] = jnp.zeros((P, G, _LANE), jnp.int32)
        spc_ref[...] = jnp.full((P, G, _LANE), _BIG, jnp.float32)
        path_ref[...] = jnp.zeros((P, G, _LANE), jnp.int32)
        minv_ref[...] = jnp.zeros((P, 1), jnp.float32)

        def sp_cond(s):
            m = s[P]
            for p in range(1, P):
                m = jnp.minimum(m, s[P + p])
            return m < 0

        def sp_body(s):
            A = []
            for p in range(P):
                cur, sink = s[p], s[P + p]
                active = sink < 0
                curmask = (riota == cur) & active
                ucur = jnp.sum(jnp.where(curmask, u_ref[p:p + 1, :], 0.0),
                               axis=1, keepdims=True)
                c4rcur = jnp.sum(jnp.where(curmask, c4r_ref[p:p + 1, :], 0),
                                 axis=1, keepdims=True)
                crow = cost_ref[p, cur]
                A.append((cur, sink, active, curmask, ucur, c4rcur, crow))
            B = []
            for p in range(P):
                cur, sink, active, curmask, ucur, c4rcur, crow = A[p]
                minv = minv_ref[p:p + 1, :]
                pathpk = (cur + 1) * 4096 + (c4rcur + 1)
                red = ((minv + crow) - ucur) - v_ref[p]
                sc = sc_ref[p]
                spc = spc_ref[p]
                better = active & (sc == 0) & (red < spc)
                spc_new = jnp.where(better, red, spc)
                path_new = jnp.where(better, pathpk, path_ref[p])
                masked = jnp.where(sc != 0, _BIG, spc_new)
                minv_row = jnp.min(masked, axis=0, keepdims=True)
                minv_new = jnp.min(minv_row, axis=1, keepdims=True)
                B.append((sc, masked, minv_new, spc_new, path_new, minv))
            K = []
            for p in range(P):
                sc, masked, minv_new, spc_new, path_new, minv = B[p]
                key = jnp.where(masked == minv_new,
                                fiota * 256 + (r4c_ref[p] + 1),
                                jnp.int32(1 << 30))
                kmin = jnp.min(jnp.min(key, axis=0, keepdims=True))
                K.append(kmin)
            new_cur, new_sink, stores = [], [], []
            for p in range(P):
                cur, sink, active, curmask, ucur, c4rcur, crow = A[p]
                sc, masked, minv_new, spc_new, path_new, minv = B[p]
                kmin = K[p]
                j = kmin >> 8
                r4cj = (kmin & 255) - 1
                sr_new = jnp.where(curmask, 1, sr_ref[p:p + 1, :])
                sc_new = jnp.where(active & (fiota == j), 1, sc)
                unassigned = r4cj < 0
                new_sink.append(jnp.where(active,
                                          jnp.where(unassigned, j, jnp.int32(-1)),
                                          sink))
                new_cur.append(jnp.where(active & ~unassigned, r4cj, cur))
                rowminv_new = jnp.where(
                    (riota == r4cj) & active, minv_new, rowminv_ref[p:p + 1, :])
                minv_keep = jnp.where(active, minv_new, minv)
                stores.append((sr_new, spc_new, path_new, sc_new,
                               rowminv_new, minv_keep))
            for p in range(P):
                sr_new, spc_new, path_new, sc_new, rowminv_new, minv_keep = stores[p]
                sr_ref[p:p + 1, :] = sr_new
                spc_ref[p] = spc_new
                path_ref[p] = path_new
                sc_ref[p] = sc_new
                rowminv_ref[p:p + 1, :] = rowminv_new
                minv_ref[p:p + 1, :] = minv_keep
            return (*new_cur, *new_sink)

        init = tuple([i] * P) + tuple([jnp.int32(-1)] * P)
        fin = jax.lax.while_loop(sp_cond, sp_body, init)
        sinks = fin[P:]

        minv_all = minv_ref[...]
        u = u_ref[...]
        u = jnp.where(riota == i, u + minv_all, u)
        rmask = (sr_ref[...] != 0) & (riota != i)
        u_ref[...] = jnp.where(rmask, (u + minv_all) - rowminv_ref[...], u)
        v_new = [jnp.where(sc_ref[p] != 0,
                           (v_ref[p] + spc_ref[p]) - minv_ref[p:p + 1, :],
                           v_ref[p]) for p in range(P)]
        for p in range(P):
            v_ref[p] = v_new[p]

        def aug_cond(s):
            m = s[P]
            for p in range(1, P):
                m = jnp.minimum(m, s[P + p])
            return m == 0

        def aug_body(s):
            pks = []
            for p in range(P):
                j, done = s[p], s[P + p]
                active = done == 0
                pkrow = jnp.sum(jnp.where((fiota == j) & active,
                                          path_ref[p], 0),
                                axis=0, keepdims=True)
                pks.append((j, done, active, jnp.sum(pkrow)))
            new_j, new_done, stores = [], [], []
            for p in range(P):
                j, done, active, pk = pks[p]
                ii = (pk >> 12) - 1
                nj = (pk & 4095) - 1
                r4c_new = jnp.where(active & (fiota == j), ii, r4c_ref[p])
                c4r_new = jnp.where((riota == ii) & active, j,
                                    c4r_ref[p:p + 1, :])
                c4rcol_new = jnp.where((citer == ii) & active, j,
                                       c4rcol_ref[p])
                new_j.append(jnp.where(active, nj, j))
                new_done.append(jnp.where(active & (ii != i), jnp.int32(0),
                                          jnp.int32(1)))
                stores.append((r4c_new, c4r_new, c4rcol_new))
            for p in range(P):
                r4c_new, c4r_new, c4rcol_new = stores[p]
                r4c_ref[p] = r4c_new
                c4r_ref[p:p + 1, :] = c4r_new
                c4rcol_ref[p] = c4rcol_new
            return (*new_j, *new_done)

        jax.lax.while_loop(aug_cond, aug_body,
                           tuple(sinks) + tuple([jnp.int32(0)] * P))
        return 0

    jax.lax.fori_loop(0, M, outer, 0)

    for p in range(P):
        c4r = c4r_ref[p:p + 1, :]
        c4rcol = c4rcol_ref[p]
        less = c4r < c4rcol
        rank = jnp.sum(less.astype(jnp.int32), axis=1, keepdims=True)
        onehot = rank == riota
        pred_sorted = jnp.sum(jnp.where(onehot, c4rcol, 0), axis=0, keepdims=True)
        tgt_sorted = jnp.sum(jnp.where(onehot, citer, 0), axis=0, keepdims=True)
        out_ref[p, 0:1, :] = pred_sorted
        out_ref[p, 1:2, :] = tgt_sorted


def kernel(out_boxes, tgt_boxes):
    B, N, _ = out_boxes.shape
    M = tgt_boxes.shape[1]
    G = N // _LANE
    pred = jnp.transpose(out_boxes, (0, 2, 1)).reshape(B, 4, G, _LANE)
    pred = jnp.pad(pred, ((0, 0), (0, 4), (0, 0), (0, 0)))
    tgt = jnp.pad(tgt_boxes, ((0, 0), (0, 0), (0, 4)))
    out = pl.pallas_call(
        _matcher_body,
        grid=(B // _P,),
        in_specs=[pl.BlockSpec((_P, 8, G, _LANE), lambda b: (b, 0, 0, 0)),
                  pl.BlockSpec((_P, M, 8), lambda b: (b, 0, 0))],
        out_specs=pl.BlockSpec((_P, 8, M), lambda b: (b, 0, 0)),
        out_shape=jax.ShapeDtypeStruct((B, 8, M), jnp.int32),
        scratch_shapes=[pltpu.VMEM((_P, M, G, _LANE), jnp.float32),
                        pltpu.VMEM((_P, M), jnp.float32),
                        pltpu.VMEM((_P, G, _LANE), jnp.float32),
                        pltpu.VMEM((_P, G, _LANE), jnp.int32),
                        pltpu.VMEM((_P, M), jnp.int32),
                        pltpu.VMEM((_P, M, 1), jnp.int32),
                        pltpu.VMEM((_P, M), jnp.int32),
                        pltpu.VMEM((_P, M), jnp.float32),
                        pltpu.VMEM((_P, G, _LANE), jnp.int32),
                        pltpu.VMEM((_P, G, _LANE), jnp.float32),
                        pltpu.VMEM((_P, G, _LANE), jnp.int32),
                        pltpu.VMEM((_P, 1), jnp.float32)],
        compiler_params=pltpu.CompilerParams(
            dimension_semantics=("parallel",)),
    )(pred, tgt)
    return out[:, :2, :]


# speedup vs baseline: 3.8515x; 1.1623x over previous
import jax
import jax.numpy as jnp
from jax.experimental import pallas as pl
from jax.experimental.pallas import tpu as pltpu

_BIG = 1e30
_LANE = 128
_P = 16


def _matcher_body(pred_ref, tgt_ref, out_ref, cost_ref,
                  u_ref, v_ref, r4c_ref, c4r_ref, c4rcol_ref,
                  sr_ref, rowminv_ref, sc_ref, spc_ref, path_ref, minv_ref):
    P = cost_ref.shape[0]
    G = cost_ref.shape[2]
    M = cost_ref.shape[1]

    for p in range(P):
        tx1 = tgt_ref[p, :, 0:1]
        ty1 = tgt_ref[p, :, 1:2]
        tx2 = tgt_ref[p, :, 2:3]
        ty2 = tgt_ref[p, :, 3:4]
        area_t = (tx2 - tx1) * (ty2 - ty1)
        for g in range(G):
            px1 = pred_ref[p, 0, g, :]
            py1 = pred_ref[p, 1, g, :]
            px2 = pred_ref[p, 2, g, :]
            py2 = pred_ref[p, 3, g, :]
            c_l1 = ((jnp.abs(px1 - tx1) + jnp.abs(py1 - ty1))
                    + jnp.abs(px2 - tx2)) + jnp.abs(py2 - ty2)
            area_p = (px2 - px1) * (py2 - py1)
            ltx = jnp.maximum(px1, tx1)
            lty = jnp.maximum(py1, ty1)
            rbx = jnp.minimum(px2, tx2)
            rby = jnp.minimum(py2, ty2)
            iw = jnp.maximum(rbx - ltx, 0.0)
            ih = jnp.maximum(rby - lty, 0.0)
            inter = iw * ih
            union = (area_p + area_t) - inter
            iou = inter / (union + 1e-8)
            elx = jnp.minimum(px1, tx1)
            ely = jnp.minimum(py1, ty1)
            erx = jnp.maximum(px2, tx2)
            ery = jnp.maximum(py2, ty2)
            ew = jnp.maximum(erx - elx, 0.0)
            eh = jnp.maximum(ery - ely, 0.0)
            earea = ew * eh
            giou = iou - (earea - union) / (earea + 1e-8)
            cost_ref[p, :, g, :] = c_l1 - giou

    riota = jax.lax.broadcasted_iota(jnp.int32, (1, M), 1)
    citer = jax.lax.broadcasted_iota(jnp.int32, (M, 1), 0)
    fiota = (jax.lax.broadcasted_iota(jnp.int32, (G, _LANE), 0) * _LANE
             + jax.lax.broadcasted_iota(jnp.int32, (G, _LANE), 1))

    u_ref[...] = jnp.zeros((P, M), jnp.float32)
    v_ref[...] = jnp.zeros((P, G, _LANE), jnp.float32)
    r4c_ref[...] = jnp.full((P, G, _LANE), -1, jnp.int32)
    c4r_ref[...] = jnp.full((P, M), -1, jnp.int32)
    c4rcol_ref[...] = jnp.full((P, M, 1), -1, jnp.int32)

    def outer(i, _):
        sr_ref[...] = jnp.zeros((P, M), jnp.int32)
        rowminv_ref[...] = jnp.zeros((P, M), jnp.float32)
        sc_ref[...] = jnp.zeros((P, G, _LANE), jnp.int32)
        spc_ref[...] = jnp.full((P, G, _LANE), _BIG, jnp.float32)
        path_ref[...] = jnp.zeros((P, G, _LANE), jnp.int32)
        minv_ref[...] = jnp.zeros((P, 1), jnp.float32)

        def sp_cond(s):
            m = s[P]
            for p in range(1, P):
                m = jnp.minimum(m, s[P + p])
            return m < 0

        def sp_body(s):
            A = []
            for p in range(P):
                cur, sink = s[p], s[P + p]
                active = sink < 0
                curmask = (riota == cur) & active
                ucur = jnp.sum(jnp.where(curmask, u_ref[p:p + 1, :], 0.0),
                               axis=1, keepdims=True)
                c4rcur = jnp.sum(jnp.where(curmask, c4r_ref[p:p + 1, :], 0),
                                 axis=1, keepdims=True)
                crow = cost_ref[p, cur]
                A.append((cur, sink, active, curmask, ucur, c4rcur, crow))
            B = []
            for p in range(P):
                cur, sink, active, curmask, ucur, c4rcur, crow = A[p]
                minv = minv_ref[p:p + 1, :]
                pathpk = (cur + 1) * 4096 + (c4rcur + 1)
                red = ((minv + crow) - ucur) - v_ref[p]
                sc = sc_ref[p]
                spc = spc_ref[p]
                better = active & (sc == 0) & (red < spc)
                spc_new = jnp.where(better, red, spc)
                path_new = jnp.where(better, pathpk, path_ref[p])
                masked = jnp.where(sc != 0, _BIG, spc_new)
                minv_row = jnp.min(masked, axis=0, keepdims=True)
                minv_new = jnp.min(minv_row, axis=1, keepdims=True)
                B.append((sc, masked, minv_new, spc_new, path_new, minv))
            K = []
            for p in range(P):
                sc, masked, minv_new, spc_new, path_new, minv = B[p]
                key = jnp.where(masked == minv_new,
                                fiota * 256 + (r4c_ref[p] + 1),
                                jnp.int32(1 << 30))
                kmin = jnp.min(jnp.min(key, axis=0, keepdims=True))
                K.append(kmin)
            new_cur, new_sink, stores = [], [], []
            for p in range(P):
                cur, sink, active, curmask, ucur, c4rcur, crow = A[p]
                sc, masked, minv_new, spc_new, path_new, minv = B[p]
                kmin = K[p]
                j = kmin >> 8
                r4cj = (kmin & 255) - 1
                sr_new = jnp.where(curmask, 1, sr_ref[p:p + 1, :])
                sc_new = jnp.where(active & (fiota == j), 1, sc)
                unassigned = r4cj < 0
                new_sink.append(jnp.where(active,
                                          jnp.where(unassigned, j, jnp.int32(-1)),
                                          sink))
                new_cur.append(jnp.where(active & ~unassigned, r4cj, cur))
                rowminv_new = jnp.where(
                    (riota == r4cj) & active, minv_new, rowminv_ref[p:p + 1, :])
                minv_keep = jnp.where(active, minv_new, minv)
                stores.append((sr_new, spc_new, path_new, sc_new,
                               rowminv_new, minv_keep))
            for p in range(P):
                sr_new, spc_new, path_new, sc_new, rowminv_new, minv_keep = stores[p]
                sr_ref[p:p + 1, :] = sr_new
                spc_ref[p] = spc_new
                path_ref[p] = path_new
                sc_ref[p] = sc_new
                rowminv_ref[p:p + 1, :] = rowminv_new
                minv_ref[p:p + 1, :] = minv_keep
            return (*new_cur, *new_sink)

        init = tuple([i] * P) + tuple([jnp.int32(-1)] * P)
        fin = jax.lax.while_loop(sp_cond, sp_body, init)
        sinks = fin[P:]

        minv_all = minv_ref[...]
        u = u_ref[...]
        u = jnp.where(riota == i, u + minv_all, u)
        rmask = (sr_ref[...] != 0) & (riota != i)
        u_ref[...] = jnp.where(rmask, (u + minv_all) - rowminv_ref[...], u)
        v_new = [jnp.where(sc_ref[p] != 0,
                           (v_ref[p] + spc_ref[p]) - minv_ref[p:p + 1, :],
                           v_ref[p]) for p in range(P)]
        for p in range(P):
            v_ref[p] = v_new[p]

        def aug_cond(s):
            m = s[P]
            for p in range(1, P):
                m = jnp.minimum(m, s[P + p])
            return m == 0

        def aug_body(s):
            pks = []
            for p in range(P):
                j, done = s[p], s[P + p]
                active = done == 0
                pkrow = jnp.sum(jnp.where((fiota == j) & active,
                                          path_ref[p], 0),
                                axis=0, keepdims=True)
                pks.append((j, done, active, jnp.sum(pkrow)))
            new_j, new_done, stores = [], [], []
            for p in range(P):
                j, done, active, pk = pks[p]
                ii = (pk >> 12) - 1
                nj = (pk & 4095) - 1
                r4c_new = jnp.where(active & (fiota == j), ii, r4c_ref[p])
                c4r_new = jnp.where((riota == ii) & active, j,
                                    c4r_ref[p:p + 1, :])
                c4rcol_new = jnp.where((citer == ii) & active, j,
                                       c4rcol_ref[p])
                new_j.append(jnp.where(active, nj, j))
                new_done.append(jnp.where(active & (ii != i), jnp.int32(0),
                                          jnp.int32(1)))
                stores.append((r4c_new, c4r_new, c4rcol_new))
            for p in range(P):
                r4c_new, c4r_new, c4rcol_new = stores[p]
                r4c_ref[p] = r4c_new
                c4r_ref[p:p + 1, :] = c4r_new
                c4rcol_ref[p] = c4rcol_new
            return (*new_j, *new_done)

        jax.lax.while_loop(aug_cond, aug_body,
                           tuple(sinks) + tuple([jnp.int32(0)] * P))
        return 0

    jax.lax.fori_loop(0, M, outer, 0)

    for p in range(P):
        c4r = c4r_ref[p:p + 1, :]
        c4rcol = c4rcol_ref[p]
        less = c4r < c4rcol
        rank = jnp.sum(less.astype(jnp.int32), axis=1, keepdims=True)
        onehot = rank == riota
        pred_sorted = jnp.sum(jnp.where(onehot, c4rcol, 0), axis=0, keepdims=True)
        tgt_sorted = jnp.sum(jnp.where(onehot, citer, 0), axis=0, keepdims=True)
        out_ref[p, 0:1, :] = pred_sorted
        out_ref[p, 1:2, :] = tgt_sorted


def kernel(out_boxes, tgt_boxes):
    B, N, _ = out_boxes.shape
    M = tgt_boxes.shape[1]
    G = N // _LANE
    pred = jnp.transpose(out_boxes, (0, 2, 1)).reshape(B, 4, G, _LANE)
    pred = jnp.pad(pred, ((0, 0), (0, 4), (0, 0), (0, 0)))
    tgt = jnp.pad(tgt_boxes, ((0, 0), (0, 0), (0, 4)))
    out = pl.pallas_call(
        _matcher_body,
        grid=(B // _P,),
        in_specs=[pl.BlockSpec((_P, 8, G, _LANE), lambda b: (b, 0, 0, 0)),
                  pl.BlockSpec((_P, M, 8), lambda b: (b, 0, 0))],
        out_specs=pl.BlockSpec((_P, 8, M), lambda b: (b, 0, 0)),
        out_shape=jax.ShapeDtypeStruct((B, 8, M), jnp.int32),
        scratch_shapes=[pltpu.VMEM((_P, M, G, _LANE), jnp.float32),
                        pltpu.VMEM((_P, M), jnp.float32),
                        pltpu.VMEM((_P, G, _LANE), jnp.float32),
                        pltpu.VMEM((_P, G, _LANE), jnp.int32),
                        pltpu.VMEM((_P, M), jnp.int32),
                        pltpu.VMEM((_P, M, 1), jnp.int32),
                        pltpu.VMEM((_P, M), jnp.int32),
                        pltpu.VMEM((_P, M), jnp.float32),
                        pltpu.VMEM((_P, G, _LANE), jnp.int32),
                        pltpu.VMEM((_P, G, _LANE), jnp.float32),
                        pltpu.VMEM((_P, G, _LANE), jnp.int32),
                        pltpu.VMEM((_P, 1), jnp.float32)],
        compiler_params=pltpu.CompilerParams(
            dimension_semantics=("parallel",)),
    )(pred, tgt)
    return out[:, :2, :]


# speedup vs baseline: 4.4774x; 1.1625x over previous
import jax
import jax.numpy as jnp
from jax.experimental import pallas as pl
from jax.experimental.pallas import tpu as pltpu

_BIG = 1e30
_LANE = 128
_P = 16


def _matcher_body(pred_ref, tgt_ref, out_ref, cost_ref,
                  u_ref, v_ref, r4c_ref, c4r_ref,
                  sr_ref, rowminv_ref, sc_ref, spc_ref, path_ref, minv_ref):
    P = cost_ref.shape[0]
    G = cost_ref.shape[2]
    M = cost_ref.shape[1]

    for p in range(P):
        tx1 = tgt_ref[p, :, 0:1]
        ty1 = tgt_ref[p, :, 1:2]
        tx2 = tgt_ref[p, :, 2:3]
        ty2 = tgt_ref[p, :, 3:4]
        area_t = (tx2 - tx1) * (ty2 - ty1)
        for g in range(G):
            px1 = pred_ref[p, 0, g, :]
            py1 = pred_ref[p, 1, g, :]
            px2 = pred_ref[p, 2, g, :]
            py2 = pred_ref[p, 3, g, :]
            c_l1 = ((jnp.abs(px1 - tx1) + jnp.abs(py1 - ty1))
                    + jnp.abs(px2 - tx2)) + jnp.abs(py2 - ty2)
            area_p = (px2 - px1) * (py2 - py1)
            ltx = jnp.maximum(px1, tx1)
            lty = jnp.maximum(py1, ty1)
            rbx = jnp.minimum(px2, tx2)
            rby = jnp.minimum(py2, ty2)
            iw = jnp.maximum(rbx - ltx, 0.0)
            ih = jnp.maximum(rby - lty, 0.0)
            inter = iw * ih
            union = (area_p + area_t) - inter
            iou = inter / (union + 1e-8)
            elx = jnp.minimum(px1, tx1)
            ely = jnp.minimum(py1, ty1)
            erx = jnp.maximum(px2, tx2)
            ery = jnp.maximum(py2, ty2)
            ew = jnp.maximum(erx - elx, 0.0)
            eh = jnp.maximum(ery - ely, 0.0)
            earea = ew * eh
            giou = iou - (earea - union) / (earea + 1e-8)
            cost_ref[p, :, g, :] = c_l1 - giou

    riota = jax.lax.broadcasted_iota(jnp.int32, (1, M), 1)
    citer = jax.lax.broadcasted_iota(jnp.int32, (M, 1), 0)
    fiota = (jax.lax.broadcasted_iota(jnp.int32, (G, _LANE), 0) * _LANE
             + jax.lax.broadcasted_iota(jnp.int32, (G, _LANE), 1))

    u_ref[...] = jnp.zeros((P, M), jnp.float32)
    v_ref[...] = jnp.zeros((P, G, _LANE), jnp.float32)
    r4c_ref[...] = jnp.full((P, G, _LANE), -1, jnp.int32)
    c4r_ref[...] = jnp.full((P, M), -1, jnp.int32)

    def outer(i, _):
        sr_ref[...] = jnp.zeros((P, M), jnp.int32)
        sc_ref[...] = jnp.zeros((P, G, _LANE), jnp.int32)
        spc_ref[...] = jnp.full((P, G, _LANE), _BIG, jnp.float32)
        minv_ref[...] = jnp.zeros((P, 1), jnp.float32)

        def _treemin(vals):
            vals = list(vals)
            while len(vals) > 1:
                vals = [jnp.minimum(vals[k], vals[k + 1])
                        if k + 1 < len(vals) else vals[k]
                        for k in range(0, len(vals), 2)]
            return vals[0]

        def sp_cond(s):
            return _treemin(s[P:]) < 0

        def sp_body(s):
            A = []
            for p in range(P):
                cur, sink = s[p], s[P + p]
                active = sink < 0
                curmask = (riota == cur) & active
                ucur = jnp.sum(jnp.where(curmask, u_ref[p:p + 1, :], 0.0),
                               axis=1, keepdims=True)
                c4rcur = jnp.sum(jnp.where(curmask, c4r_ref[p:p + 1, :], 0),
                                 axis=1, keepdims=True)
                crow = cost_ref[p, cur]
                A.append((cur, sink, active, curmask, ucur, c4rcur, crow))
            B = []
            for p in range(P):
                cur, sink, active, curmask, ucur, c4rcur, crow = A[p]
                minv = minv_ref[p:p + 1, :]
                pathpk = (cur + 1) * 4096 + (c4rcur + 1)
                red = ((minv + crow) - ucur) - v_ref[p]
                sc = sc_ref[p]
                spc = spc_ref[p]
                better = active & (sc == 0) & (red < spc)
                spc_new = jnp.where(better, red, spc)
                path_new = jnp.where(better, pathpk, path_ref[p])
                masked = jnp.where(sc != 0, _BIG, spc_new)
                minv_row = jnp.min(masked, axis=0, keepdims=True)
                minv_new = jnp.min(minv_row, axis=1, keepdims=True)
                B.append((sc, masked, minv_new, spc_new, path_new, minv))
            K = []
            for p in range(P):
                sc, masked, minv_new, spc_new, path_new, minv = B[p]
                key = jnp.where(masked == minv_new,
                                fiota * 256 + (r4c_ref[p] + 1),
                                jnp.int32(1 << 30))
                kmin = jnp.min(jnp.min(key, axis=0, keepdims=True))
                K.append(kmin)
            new_cur, new_sink, stores = [], [], []
            for p in range(P):
                cur, sink, active, curmask, ucur, c4rcur, crow = A[p]
                sc, masked, minv_new, spc_new, path_new, minv = B[p]
                kmin = K[p]
                j = kmin >> 8
                r4cj = (kmin & 255) - 1
                sr_new = jnp.where(curmask, 1, sr_ref[p:p + 1, :])
                sc_new = jnp.where(active & (fiota == j), 1, sc)
                unassigned = r4cj < 0
                new_sink.append(jnp.where(active,
                                          jnp.where(unassigned, j, jnp.int32(-1)),
                                          sink))
                new_cur.append(jnp.where(active & ~unassigned, r4cj, cur))
                rowminv_new = jnp.where(
                    (riota == r4cj) & active, minv_new, rowminv_ref[p:p + 1, :])
                minv_keep = jnp.where(active, minv_new, minv)
                stores.append((sr_new, spc_new, path_new, sc_new,
                               rowminv_new, minv_keep))
            for p in range(P):
                sr_new, spc_new, path_new, sc_new, rowminv_new, minv_keep = stores[p]
                sr_ref[p:p + 1, :] = sr_new
                spc_ref[p] = spc_new
                path_ref[p] = path_new
                sc_ref[p] = sc_new
                rowminv_ref[p:p + 1, :] = rowminv_new
                minv_ref[p:p + 1, :] = minv_keep
            return (*new_cur, *new_sink)

        init = tuple([i] * P) + tuple([jnp.int32(-1)] * P)
        fin = jax.lax.while_loop(sp_cond, sp_body, init)
        sinks = fin[P:]

        minv_all = minv_ref[...]
        u = u_ref[...]
        u = jnp.where(riota == i, u + minv_all, u)
        rmask = (sr_ref[...] != 0) & (riota != i)
        u_ref[...] = jnp.where(rmask, (u + minv_all) - rowminv_ref[...], u)
        v_new = [jnp.where(sc_ref[p] != 0,
                           (v_ref[p] + spc_ref[p]) - minv_ref[p:p + 1, :],
                           v_ref[p]) for p in range(P)]
        for p in range(P):
            v_ref[p] = v_new[p]

        def aug_cond(s):
            return _treemin(s[P:]) == 0

        def aug_body(s):
            pks = []
            for p in range(P):
                j, done = s[p], s[P + p]
                active = done == 0
                pkrow = jnp.sum(jnp.where((fiota == j) & active,
                                          path_ref[p], 0),
                                axis=0, keepdims=True)
                pks.append((j, done, active, jnp.sum(pkrow)))
            new_j, new_done, stores = [], [], []
            for p in range(P):
                j, done, active, pk = pks[p]
                ii = (pk >> 12) - 1
                nj = (pk & 4095) - 1
                r4c_new = jnp.where(active & (fiota == j), ii, r4c_ref[p])
                c4r_new = jnp.where((riota == ii) & active, j,
                                    c4r_ref[p:p + 1, :])
                new_j.append(jnp.where(active, nj, j))
                new_done.append(jnp.where(active & (ii != i), jnp.int32(0),
                                          jnp.int32(1)))
                stores.append((r4c_new, c4r_new))
            for p in range(P):
                r4c_new, c4r_new = stores[p]
                r4c_ref[p] = r4c_new
                c4r_ref[p:p + 1, :] = c4r_new
            return (*new_j, *new_done)

        jax.lax.while_loop(aug_cond, aug_body,
                           tuple(sinks) + tuple([jnp.int32(0)] * P))
        return 0

    jax.lax.fori_loop(0, M, outer, 0)

    for p in range(P):
        c4r = c4r_ref[p:p + 1, :]
        c4rcol = jnp.sum(jnp.where(riota == citer, c4r, 0),
                         axis=1, keepdims=True)
        less = c4r < c4rcol
        rank = jnp.sum(less.astype(jnp.int32), axis=1, keepdims=True)
        onehot = rank == riota
        pred_sorted = jnp.sum(jnp.where(onehot, c4rcol, 0), axis=0, keepdims=True)
        tgt_sorted = jnp.sum(jnp.where(onehot, citer, 0), axis=0, keepdims=True)
        out_ref[p, 0:1, :] = pred_sorted
        out_ref[p, 1:2, :] = tgt_sorted


def kernel(out_boxes, tgt_boxes):
    B, N, _ = out_boxes.shape
    M = tgt_boxes.shape[1]
    G = N // _LANE
    pred = jnp.transpose(out_boxes, (0, 2, 1)).reshape(B, 4, G, _LANE)
    pred = jnp.pad(pred, ((0, 0), (0, 4), (0, 0), (0, 0)))
    tgt = jnp.pad(tgt_boxes, ((0, 0), (0, 0), (0, 4)))
    out = pl.pallas_call(
        _matcher_body,
        grid=(B // _P,),
        in_specs=[pl.BlockSpec((_P, 8, G, _LANE), lambda b: (b, 0, 0, 0)),
                  pl.BlockSpec((_P, M, 8), lambda b: (b, 0, 0))],
        out_specs=pl.BlockSpec((_P, 8, M), lambda b: (b, 0, 0)),
        out_shape=jax.ShapeDtypeStruct((B, 8, M), jnp.int32),
        scratch_shapes=[pltpu.VMEM((_P, M, G, _LANE), jnp.float32),
                        pltpu.VMEM((_P, M), jnp.float32),
                        pltpu.VMEM((_P, G, _LANE), jnp.float32),
                        pltpu.VMEM((_P, G, _LANE), jnp.int32),
                        pltpu.VMEM((_P, M), jnp.int32),
                        pltpu.VMEM((_P, M), jnp.int32),
                        pltpu.VMEM((_P, M), jnp.float32),
                        pltpu.VMEM((_P, G, _LANE), jnp.int32),
                        pltpu.VMEM((_P, G, _LANE), jnp.float32),
                        pltpu.VMEM((_P, G, _LANE), jnp.int32),
                        pltpu.VMEM((_P, 1), jnp.float32)],
        compiler_params=pltpu.CompilerParams(
            dimension_semantics=("parallel",)),
    )(pred, tgt)
    return out[:, :2, :]
